# Initial kernel scaffold; baseline (speedup 1.0000x reference)
import jax
import jax.numpy as jnp
from jax.experimental import pallas as pl


def kernel(node_features, edge_index, edge_attr, post_mask, W_node, b_node, W_edge, b_edge, W_conv, b_conv, W_out, b_out):
    raise NotImplementedError("write your pallas kernel here")


# speedup vs baseline: 2.6679x; 2.6679x over previous
import functools

import jax
import jax.numpy as jnp
from jax import lax
from jax.experimental import pallas as pl
from jax.experimental.pallas import tpu as pltpu
from jax.experimental.pallas import tpu_sc as plsc


def _node_body(x_ref, wnt_ref, bn_ref, a_ref, ne_ref, tn_ref):
    h = jnp.dot(x_ref[...], wnt_ref[...], preferred_element_type=jnp.float32)
    h = jnp.maximum(h + bn_ref[...], 0.0)
    ne_ref[...] = h
    tn_ref[...] = jnp.dot(h, a_ref[...], preferred_element_type=jnp.float32)


def _edge_body(e_ref, wet_ref, be_ref, bmat_ref, bc_ref, te_ref):
    h = jnp.dot(e_ref[...], wet_ref[...], preferred_element_type=jnp.float32)
    h = jnp.maximum(h + be_ref[...], 0.0)
    te_ref[...] = jnp.dot(h, bmat_ref[...], preferred_element_type=jnp.float32) + bc_ref[...]


def _final_body(ne_ref, p0_ref, p1_ref, w_ref, b_ref, z_ref):
    h = ne_ref[...] + p0_ref[...] + p1_ref[...]
    z = jnp.sum(h * w_ref[...], axis=1, keepdims=True) + b_ref[...]
    z_ref[...] = jax.nn.sigmoid(z)


_NC = 2
_NS = 16
_NW = _NC * _NS
_CB = 128


def _make_scatter_kernel(n_pad, chunks):
    per_w = chunks * _CB
    rows_per_s = n_pad // _NS
    mesh = plsc.VectorSubcoreMesh(core_axis_name="c", subcore_axis_name="s",
                                  num_cores=_NC, num_subcores=_NS)

    @functools.partial(
        pl.kernel,
        out_type=jax.ShapeDtypeStruct((_NC, n_pad, 32), jnp.float32),
        mesh=mesh,
        scratch_types=[
            pltpu.VMEM((_CB,), jnp.int32),
            pltpu.VMEM((_CB,), jnp.int32),
            pltpu.VMEM((_CB, 32), jnp.float32),
            pltpu.VMEM((_CB, 32), jnp.float32),
            pltpu.VMEM_SHARED((n_pad, 32), jnp.float32),
            pltpu.SemaphoreType.DMA,
        ],
        compiler_params=pltpu.CompilerParams(use_tc_tiling_on_sc=False),
    )
    def scatter_k(src_hbm, dst_hbm, te_hbm, tn_hbm, zeros_hbm, out_hbm,
                  sidx, didx, tev, rowsv, acc, sem):
        c = lax.axis_index("c")
        s = lax.axis_index("s")
        wid = s * _NC + c
        pltpu.sync_copy(zeros_hbm.at[pl.ds(s * rows_per_s, rows_per_s)],
                        acc.at[pl.ds(s * rows_per_s, rows_per_s)])
        plsc.subcore_barrier()

        def body(j, carry):
            base = wid * per_w + j * _CB
            pltpu.sync_copy(src_hbm.at[pl.ds(base, _CB)], sidx)
            pltpu.sync_copy(dst_hbm.at[pl.ds(base, _CB)], didx)
            pltpu.async_copy(tn_hbm.at[sidx], rowsv, sem).wait()
            pltpu.sync_copy(te_hbm.at[pl.ds(base, _CB)], tev)
            pltpu.sync_copy(rowsv, acc.at[didx], add=True)
            pltpu.sync_copy(tev, acc.at[didx], add=True)
            return carry

        lax.fori_loop(0, chunks, body, 0)
        plsc.subcore_barrier()
        pltpu.sync_copy(acc.at[pl.ds(s * rows_per_s, rows_per_s)],
                        out_hbm.at[c, pl.ds(s * rows_per_s, rows_per_s)])

    return scatter_k


def _make_gather_kernel(n_nodes, p_pad):
    per_w = p_pad // _NW
    groups = per_w // 16
    mesh = plsc.VectorSubcoreMesh(core_axis_name="c", subcore_axis_name="s",
                                  num_cores=_NC, num_subcores=_NS)

    @functools.partial(
        pl.kernel,
        out_type=jax.ShapeDtypeStruct((p_pad,), jnp.float32),
        mesh=mesh,
        scratch_types=[
            pltpu.VMEM((n_nodes,), jnp.float32),
            pltpu.VMEM((per_w,), jnp.int32),
            pltpu.VMEM((per_w,), jnp.float32),
        ],
        compiler_params=pltpu.CompilerParams(needs_layout_passes=False),
    )
    def gather_k(z_hbm, pm_hbm, out_hbm, zv, idxv, outv):
        c = lax.axis_index("c")
        s = lax.axis_index("s")
        wid = s * _NC + c
        pltpu.sync_copy(z_hbm, zv)
        pltpu.sync_copy(pm_hbm.at[pl.ds(wid * per_w, per_w)], idxv)
        for g in range(groups):
            idx = idxv[pl.ds(g * 16, 16)]
            outv[pl.ds(g * 16, 16)] = plsc.load_gather(zv, [idx])
        pltpu.sync_copy(outv, out_hbm.at[pl.ds(wid * per_w, per_w)])

    return gather_k


def kernel(node_features, edge_index, edge_attr, post_mask,
           W_node, b_node, W_edge, b_edge, W_conv, b_conv, W_out, b_out):
    n, d_node = node_features.shape
    e = edge_attr.shape[0]
    d_edge = edge_attr.shape[1]
    h = W_node.shape[0]
    p = post_mask.shape[0]

    chunks = -(-e // (_NW * _CB))
    e_pad = _NW * chunks * _CB
    dust = n
    n_pad = -(-(n + 1) // (8 * _NS)) * (8 * _NS)
    p_pad = -(-p // (16 * _NW)) * (16 * _NW)

    wnt = W_node.T
    wet = W_edge.T
    a_mat = W_conv[:, :h].T
    b_mat = W_conv[:, h:].T
    bn2 = b_node.reshape(1, h)
    be2 = b_edge.reshape(1, h)
    bc2 = b_conv.reshape(1, h)
    w2 = W_out.reshape(1, h)
    bo2 = b_out.reshape(1, 1)

    src = jnp.concatenate([edge_index[0], jnp.zeros((e_pad - e,), jnp.int32)])
    dst = jnp.concatenate([edge_index[1], jnp.full((e_pad - e,), dust, jnp.int32)])
    ep = jnp.pad(edge_attr, ((0, e_pad - e), (0, 0)))
    pm = jnp.pad(post_mask, (0, p_pad - p))
    zeros_acc = jnp.zeros((n_pad, 32), jnp.float32)

    nb = 512
    ng = -(-n // nb)
    node_emb, tn = pl.pallas_call(
        _node_body,
        grid=(ng,),
        in_specs=[
            pl.BlockSpec((nb, d_node), lambda i: (i, 0)),
            pl.BlockSpec((d_node, h), lambda i: (0, 0)),
            pl.BlockSpec((1, h), lambda i: (0, 0)),
            pl.BlockSpec((h, h), lambda i: (0, 0)),
        ],
        out_specs=[pl.BlockSpec((nb, h), lambda i: (i, 0)),
                   pl.BlockSpec((nb, h), lambda i: (i, 0))],
        out_shape=[jax.ShapeDtypeStruct((n, h), jnp.float32),
                   jax.ShapeDtypeStruct((n, h), jnp.float32)],
    )(node_features, wnt, bn2, a_mat)

    eb = 2048
    eg = e_pad // eb
    te = pl.pallas_call(
        _edge_body,
        grid=(eg,),
        in_specs=[
            pl.BlockSpec((eb, d_edge), lambda i: (i, 0)),
            pl.BlockSpec((d_edge, h), lambda i: (0, 0)),
            pl.BlockSpec((1, h), lambda i: (0, 0)),
            pl.BlockSpec((h, h), lambda i: (0, 0)),
            pl.BlockSpec((1, h), lambda i: (0, 0)),
        ],
        out_specs=pl.BlockSpec((eb, h), lambda i: (i, 0)),
        out_shape=jax.ShapeDtypeStruct((e_pad, h), jnp.float32),
    )(ep, wet, be2, b_mat, bc2)

    partials = _make_scatter_kernel(n_pad, chunks)(src, dst, te, tn, zeros_acc)

    fb = 1024
    fg = -(-n_pad // fb)
    z = pl.pallas_call(
        _final_body,
        grid=(fg,),
        in_specs=[
            pl.BlockSpec((fb, h), lambda i: (i, 0)),
            pl.BlockSpec((fb, h), lambda i: (i, 0)),
            pl.BlockSpec((fb, h), lambda i: (i, 0)),
            pl.BlockSpec((1, h), lambda i: (0, 0)),
            pl.BlockSpec((1, 1), lambda i: (0, 0)),
        ],
        out_specs=pl.BlockSpec((fb, 1), lambda i: (i, 0)),
        out_shape=jax.ShapeDtypeStruct((n, 1), jnp.float32),
    )(node_emb, partials[0], partials[1], w2, bo2)

    out = _make_gather_kernel(n, p_pad)(z.reshape(n), pm)
    return out[:p]


# speedup vs baseline: 3.0320x; 1.1365x over previous
import functools

import jax
import jax.numpy as jnp
from jax import lax
from jax.experimental import pallas as pl
from jax.experimental.pallas import tpu as pltpu
from jax.experimental.pallas import tpu_sc as plsc


def _node_body(x_ref, wnt_ref, bn_ref, a_ref, ne_ref, tn_ref):
    h = jnp.dot(x_ref[...], wnt_ref[...], preferred_element_type=jnp.float32)
    h = jnp.maximum(h + bn_ref[...], 0.0)
    ne_ref[...] = h
    tn_ref[...] = jnp.dot(h, a_ref[...], preferred_element_type=jnp.float32)


def _edge_body(e_ref, wet_ref, be_ref, bmat_ref, bc_ref, te_ref):
    h = jnp.dot(e_ref[...], wet_ref[...], preferred_element_type=jnp.float32)
    h = jnp.maximum(h + be_ref[...], 0.0)
    te_ref[...] = jnp.dot(h, bmat_ref[...], preferred_element_type=jnp.float32) + bc_ref[...]


def _final_body(ne_ref, p0_ref, p1_ref, w_ref, b_ref, z_ref):
    h = ne_ref[...] + p0_ref[...] + p1_ref[...]
    z = jnp.sum(h * w_ref[...], axis=1, keepdims=True) + b_ref[...]
    z_ref[...] = jax.nn.sigmoid(z)


_NC = 2
_NS = 16
_NW = _NC * _NS
_CB = 128


_GRP = 8


def _make_scatter_kernel(n_pad, chunks):
    rows_per_s = n_pad // _NS
    mesh = plsc.VectorSubcoreMesh(core_axis_name="c", subcore_axis_name="s",
                                  num_cores=_NC, num_subcores=_NS)

    @functools.partial(
        pl.kernel,
        out_type=jax.ShapeDtypeStruct((_NC, n_pad, 32), jnp.float32),
        mesh=mesh,
        scratch_types=[
            pltpu.VMEM((chunks, _CB), jnp.int32),
            pltpu.VMEM((chunks, _CB), jnp.int32),
            pltpu.VMEM((_GRP, _CB, 32), jnp.float32),
            pltpu.VMEM((_GRP, _CB, 32), jnp.float32),
            pltpu.VMEM_SHARED((n_pad, 32), jnp.float32),
            pltpu.SemaphoreType.DMA,
            pltpu.SemaphoreType.DMA,
        ],
        compiler_params=pltpu.CompilerParams(use_tc_tiling_on_sc=False),
    )
    def scatter_k(src_hbm, dst_hbm, te_hbm, tn_hbm, zeros_hbm, out_hbm,
                  sidx, didx, tev, rowsv, acc, gsem, ssem):
        c = lax.axis_index("c")
        s = lax.axis_index("s")
        wid = s * _NC + c
        pltpu.sync_copy(zeros_hbm.at[pl.ds(s * rows_per_s, rows_per_s)],
                        acc.at[pl.ds(s * rows_per_s, rows_per_s)])
        pltpu.sync_copy(src_hbm.at[pl.ds(wid * chunks, chunks)], sidx)
        pltpu.sync_copy(dst_hbm.at[pl.ds(wid * chunks, chunks)], didx)
        plsc.subcore_barrier()

        def body(g, carry):
            j0 = g * _GRP
            loads = []
            for r in range(_GRP):
                loads.append(pltpu.async_copy(
                    tn_hbm.at[sidx.at[j0 + r]], rowsv.at[r], gsem))
                loads.append(pltpu.async_copy(
                    te_hbm.at[pl.ds((wid * chunks + j0 + r) * _CB, _CB)],
                    tev.at[r], gsem))
            for d in loads:
                d.wait()
            stores = []
            for r in range(_GRP):
                stores.append(pltpu.async_copy(
                    rowsv.at[r], acc.at[didx.at[j0 + r]], ssem, add=True))
                stores.append(pltpu.async_copy(
                    tev.at[r], acc.at[didx.at[j0 + r]], ssem, add=True))
            for d in stores:
                d.wait()
            return carry

        lax.fori_loop(0, chunks // _GRP, body, 0)
        plsc.subcore_barrier()
        pltpu.sync_copy(acc.at[pl.ds(s * rows_per_s, rows_per_s)],
                        out_hbm.at[c, pl.ds(s * rows_per_s, rows_per_s)])

    return scatter_k


def _make_gather_kernel(n_nodes, p_pad):
    per_w = p_pad // _NW
    groups = per_w // 16
    mesh = plsc.VectorSubcoreMesh(core_axis_name="c", subcore_axis_name="s",
                                  num_cores=_NC, num_subcores=_NS)

    @functools.partial(
        pl.kernel,
        out_type=jax.ShapeDtypeStruct((p_pad,), jnp.float32),
        mesh=mesh,
        scratch_types=[
            pltpu.VMEM((n_nodes,), jnp.float32),
            pltpu.VMEM((per_w,), jnp.int32),
            pltpu.VMEM((per_w,), jnp.float32),
        ],
        compiler_params=pltpu.CompilerParams(needs_layout_passes=False),
    )
    def gather_k(z_hbm, pm_hbm, out_hbm, zv, idxv, outv):
        c = lax.axis_index("c")
        s = lax.axis_index("s")
        wid = s * _NC + c
        pltpu.sync_copy(z_hbm, zv)
        pltpu.sync_copy(pm_hbm.at[pl.ds(wid * per_w, per_w)], idxv)
        for g in range(groups):
            idx = idxv[pl.ds(g * 16, 16)]
            outv[pl.ds(g * 16, 16)] = plsc.load_gather(zv, [idx])
        pltpu.sync_copy(outv, out_hbm.at[pl.ds(wid * per_w, per_w)])

    return gather_k


def kernel(node_features, edge_index, edge_attr, post_mask,
           W_node, b_node, W_edge, b_edge, W_conv, b_conv, W_out, b_out):
    n, d_node = node_features.shape
    e = edge_attr.shape[0]
    d_edge = edge_attr.shape[1]
    h = W_node.shape[0]
    p = post_mask.shape[0]

    chunks = -(-e // (_NW * _CB))
    chunks = -(-chunks // _GRP) * _GRP
    e_pad = _NW * chunks * _CB
    dust = n
    n_pad = -(-(n + 1) // (8 * _NS)) * (8 * _NS)
    p_pad = -(-p // (16 * _NW)) * (16 * _NW)

    wnt = W_node.T
    wet = W_edge.T
    a_mat = W_conv[:, :h].T
    b_mat = W_conv[:, h:].T
    bn2 = b_node.reshape(1, h)
    be2 = b_edge.reshape(1, h)
    bc2 = b_conv.reshape(1, h)
    w2 = W_out.reshape(1, h)
    bo2 = b_out.reshape(1, 1)

    src = jnp.concatenate([edge_index[0], jnp.zeros((e_pad - e,), jnp.int32)])
    dst = jnp.concatenate([edge_index[1], jnp.full((e_pad - e,), dust, jnp.int32)])
    ep = jnp.pad(edge_attr, ((0, e_pad - e), (0, 0)))
    pm = jnp.pad(post_mask, (0, p_pad - p))
    zeros_acc = jnp.zeros((n_pad, 32), jnp.float32)

    nb = 512
    ng = -(-n // nb)
    node_emb, tn = pl.pallas_call(
        _node_body,
        grid=(ng,),
        in_specs=[
            pl.BlockSpec((nb, d_node), lambda i: (i, 0)),
            pl.BlockSpec((d_node, h), lambda i: (0, 0)),
            pl.BlockSpec((1, h), lambda i: (0, 0)),
            pl.BlockSpec((h, h), lambda i: (0, 0)),
        ],
        out_specs=[pl.BlockSpec((nb, h), lambda i: (i, 0)),
                   pl.BlockSpec((nb, h), lambda i: (i, 0))],
        out_shape=[jax.ShapeDtypeStruct((n, h), jnp.float32),
                   jax.ShapeDtypeStruct((n, h), jnp.float32)],
    )(node_features, wnt, bn2, a_mat)

    eb = 2048
    eg = e_pad // eb
    te = pl.pallas_call(
        _edge_body,
        grid=(eg,),
        in_specs=[
            pl.BlockSpec((eb, d_edge), lambda i: (i, 0)),
            pl.BlockSpec((d_edge, h), lambda i: (0, 0)),
            pl.BlockSpec((1, h), lambda i: (0, 0)),
            pl.BlockSpec((h, h), lambda i: (0, 0)),
            pl.BlockSpec((1, h), lambda i: (0, 0)),
        ],
        out_specs=pl.BlockSpec((eb, h), lambda i: (i, 0)),
        out_shape=jax.ShapeDtypeStruct((e_pad, h), jnp.float32),
    )(ep, wet, be2, b_mat, bc2)

    partials = _make_scatter_kernel(n_pad, chunks)(
        src.reshape(-1, _CB), dst.reshape(-1, _CB), te, tn, zeros_acc)

    fb = 1024
    fg = -(-n_pad // fb)
    z = pl.pallas_call(
        _final_body,
        grid=(fg,),
        in_specs=[
            pl.BlockSpec((fb, h), lambda i: (i, 0)),
            pl.BlockSpec((fb, h), lambda i: (i, 0)),
            pl.BlockSpec((fb, h), lambda i: (i, 0)),
            pl.BlockSpec((1, h), lambda i: (0, 0)),
            pl.BlockSpec((1, 1), lambda i: (0, 0)),
        ],
        out_specs=pl.BlockSpec((fb, 1), lambda i: (i, 0)),
        out_shape=jax.ShapeDtypeStruct((n, 1), jnp.float32),
    )(node_emb, partials[0], partials[1], w2, bo2)

    out = _make_gather_kernel(n, p_pad)(z.reshape(n), pm)
    return out[:p]


# speedup vs baseline: 3.8171x; 1.2589x over previous
import functools

import jax
import jax.numpy as jnp
from jax import lax
from jax.experimental import pallas as pl
from jax.experimental.pallas import tpu as pltpu
from jax.experimental.pallas import tpu_sc as plsc


def _node_body(x_ref, wnt_ref, bn_ref, a_ref, ne_ref, tn_ref):
    h = jnp.dot(x_ref[...], wnt_ref[...], preferred_element_type=jnp.float32)
    h = jnp.maximum(h + bn_ref[...], 0.0)
    ne_ref[...] = h
    tn_ref[...] = jnp.dot(h, a_ref[...], preferred_element_type=jnp.float32)


def _edge_body(e_ref, wet_ref, be_ref, bmat_ref, bc_ref, te_ref):
    h = jnp.dot(e_ref[...], wet_ref[...], preferred_element_type=jnp.float32)
    h = jnp.maximum(h + be_ref[...], 0.0)
    te_ref[...] = jnp.dot(h, bmat_ref[...], preferred_element_type=jnp.float32) + bc_ref[...]


def _final_body(ne_ref, p0_ref, p1_ref, w_ref, b_ref, z_ref):
    h = ne_ref[...] + p0_ref[...] + p1_ref[...]
    z = jnp.sum(h * w_ref[...], axis=1, keepdims=True) + b_ref[...]
    z_ref[...] = jax.nn.sigmoid(z)


_NC = 2
_NS = 16
_NW = _NC * _NS
_CB = 125
_GRP = 8


def _make_scatter_kernel(n_pad, chunks):
    rows_per_s = n_pad // _NS
    mesh = plsc.VectorSubcoreMesh(core_axis_name="c", subcore_axis_name="s",
                                  num_cores=_NC, num_subcores=_NS)

    @functools.partial(
        pl.kernel,
        out_type=jax.ShapeDtypeStruct((_NC, n_pad, 32), jnp.float32),
        mesh=mesh,
        scratch_types=[
            pltpu.VMEM((chunks, _CB), jnp.int32),
            pltpu.VMEM((chunks, _CB), jnp.int32),
            pltpu.VMEM((_GRP, _CB, 32), jnp.float32),
            pltpu.VMEM((_GRP, _CB, 32), jnp.float32),
            pltpu.VMEM_SHARED((n_pad, 32), jnp.float32),
            pltpu.SemaphoreType.DMA,
            pltpu.SemaphoreType.DMA,
        ],
        compiler_params=pltpu.CompilerParams(use_tc_tiling_on_sc=False),
    )
    def scatter_k(eidx_hbm, te_hbm, tn_hbm, zeros_hbm, out_hbm,
                  sidx, didx, tev, rowsv, acc, gsem, ssem):
        c = lax.axis_index("c")
        s = lax.axis_index("s")
        wid = s * _NC + c
        pltpu.sync_copy(zeros_hbm.at[pl.ds(s * rows_per_s, rows_per_s)],
                        acc.at[pl.ds(s * rows_per_s, rows_per_s)])
        pltpu.sync_copy(eidx_hbm.at[0, pl.ds(wid * chunks, chunks)], sidx)
        pltpu.sync_copy(eidx_hbm.at[1, pl.ds(wid * chunks, chunks)], didx)
        plsc.subcore_barrier()

        def body(g, carry):
            j0 = g * _GRP
            loads = []
            for r in range(_GRP):
                loads.append(pltpu.async_copy(
                    tn_hbm.at[sidx.at[j0 + r]], rowsv.at[r], gsem))
                loads.append(pltpu.async_copy(
                    te_hbm.at[pl.ds((wid * chunks + j0 + r) * _CB, _CB)],
                    tev.at[r], gsem))
            for d in loads:
                d.wait()
            stores = []
            for r in range(_GRP):
                stores.append(pltpu.async_copy(
                    rowsv.at[r], acc.at[didx.at[j0 + r]], ssem, add=True))
                stores.append(pltpu.async_copy(
                    tev.at[r], acc.at[didx.at[j0 + r]], ssem, add=True))
            for d in stores:
                d.wait()
            return carry

        lax.fori_loop(0, chunks // _GRP, body, 0)
        plsc.subcore_barrier()
        pltpu.sync_copy(acc.at[pl.ds(s * rows_per_s, rows_per_s)],
                        out_hbm.at[c, pl.ds(s * rows_per_s, rows_per_s)])

    return scatter_k


def _make_gather_kernel(n_nodes, p_pad):
    per_w = p_pad // _NW
    groups = per_w // 16
    mesh = plsc.VectorSubcoreMesh(core_axis_name="c", subcore_axis_name="s",
                                  num_cores=_NC, num_subcores=_NS)

    @functools.partial(
        pl.kernel,
        out_type=jax.ShapeDtypeStruct((p_pad,), jnp.float32),
        mesh=mesh,
        scratch_types=[
            pltpu.VMEM((n_nodes,), jnp.float32),
            pltpu.VMEM((per_w,), jnp.int32),
            pltpu.VMEM((per_w,), jnp.float32),
        ],
        compiler_params=pltpu.CompilerParams(needs_layout_passes=False),
    )
    def gather_k(z_hbm, pm_hbm, out_hbm, zv, idxv, outv):
        c = lax.axis_index("c")
        s = lax.axis_index("s")
        wid = s * _NC + c
        pltpu.sync_copy(z_hbm, zv)
        pltpu.sync_copy(pm_hbm.at[pl.ds(wid * per_w, per_w)], idxv)
        for g in range(groups):
            idx = idxv[pl.ds(g * 16, 16)]
            outv[pl.ds(g * 16, 16)] = plsc.load_gather(zv, [idx])
        pltpu.sync_copy(outv, out_hbm.at[pl.ds(wid * per_w, per_w)])

    return gather_k


def kernel(node_features, edge_index, edge_attr, post_mask,
           W_node, b_node, W_edge, b_edge, W_conv, b_conv, W_out, b_out):
    n, d_node = node_features.shape
    e = edge_attr.shape[0]
    d_edge = edge_attr.shape[1]
    h = W_node.shape[0]
    p = post_mask.shape[0]

    chunks = e // (_NW * _CB)
    n_pad = -(-(n + 1) // (8 * _NS)) * (8 * _NS)
    p_pad = -(-p // (16 * _NW)) * (16 * _NW)

    wnt = W_node.T
    wet = W_edge.T
    a_mat = W_conv[:, :h].T
    b_mat = W_conv[:, h:].T
    bn2 = b_node.reshape(1, h)
    be2 = b_edge.reshape(1, h)
    bc2 = b_conv.reshape(1, h)
    w2 = W_out.reshape(1, h)
    bo2 = b_out.reshape(1, 1)

    eidx = edge_index.reshape(2, _NW * chunks, _CB)
    pm = jnp.pad(post_mask, (0, p_pad - p))
    zeros_acc = jnp.zeros((n_pad, 32), jnp.float32)

    nb = 512
    ng = -(-n // nb)
    node_emb, tn = pl.pallas_call(
        _node_body,
        grid=(ng,),
        in_specs=[
            pl.BlockSpec((nb, d_node), lambda i: (i, 0)),
            pl.BlockSpec((d_node, h), lambda i: (0, 0)),
            pl.BlockSpec((1, h), lambda i: (0, 0)),
            pl.BlockSpec((h, h), lambda i: (0, 0)),
        ],
        out_specs=[pl.BlockSpec((nb, h), lambda i: (i, 0)),
                   pl.BlockSpec((nb, h), lambda i: (i, 0))],
        out_shape=[jax.ShapeDtypeStruct((n, h), jnp.float32),
                   jax.ShapeDtypeStruct((n, h), jnp.float32)],
    )(node_features, wnt, bn2, a_mat)

    eb = 2048
    eg = -(-e // eb)
    te = pl.pallas_call(
        _edge_body,
        grid=(eg,),
        in_specs=[
            pl.BlockSpec((eb, d_edge), lambda i: (i, 0)),
            pl.BlockSpec((d_edge, h), lambda i: (0, 0)),
            pl.BlockSpec((1, h), lambda i: (0, 0)),
            pl.BlockSpec((h, h), lambda i: (0, 0)),
            pl.BlockSpec((1, h), lambda i: (0, 0)),
        ],
        out_specs=pl.BlockSpec((eb, h), lambda i: (i, 0)),
        out_shape=jax.ShapeDtypeStruct((e, h), jnp.float32),
    )(edge_attr, wet, be2, b_mat, bc2)

    partials = _make_scatter_kernel(n_pad, chunks)(eidx, te, tn, zeros_acc)

    fb = 1024
    fg = -(-n_pad // fb)
    z = pl.pallas_call(
        _final_body,
        grid=(fg,),
        in_specs=[
            pl.BlockSpec((fb, h), lambda i: (i, 0)),
            pl.BlockSpec((fb, h), lambda i: (i, 0)),
            pl.BlockSpec((fb, h), lambda i: (i, 0)),
            pl.BlockSpec((1, h), lambda i: (0, 0)),
            pl.BlockSpec((1, 1), lambda i: (0, 0)),
        ],
        out_specs=pl.BlockSpec((fb, 1), lambda i: (i, 0)),
        out_shape=jax.ShapeDtypeStruct((n, 1), jnp.float32),
    )(node_emb, partials[0], partials[1], w2, bo2)

    out = _make_gather_kernel(n, p_pad)(z.reshape(n), pm)
    return out[:p]


# speedup vs baseline: 6.5217x; 1.7086x over previous
import functools

import jax
import jax.numpy as jnp
from jax import lax
from jax.experimental import pallas as pl
from jax.experimental.pallas import tpu as pltpu
from jax.experimental.pallas import tpu_sc as plsc


def _node_body(x_ref, wnt_ref, bn_ref, a_ref, ne_ref, tn_ref):
    h = jnp.dot(x_ref[...], wnt_ref[...], preferred_element_type=jnp.float32)
    h = jnp.maximum(h + bn_ref[...], 0.0)
    ne_ref[...] = h
    tn_ref[...] = jnp.dot(h, a_ref[...], preferred_element_type=jnp.float32)


def _edge_body(e_ref, wet_ref, be_ref, bmat_ref, bc_ref, te_ref):
    h = jnp.dot(e_ref[...], wet_ref[...], preferred_element_type=jnp.float32)
    h = jnp.maximum(h + be_ref[...], 0.0)
    t = jnp.dot(h, bmat_ref[...], preferred_element_type=jnp.float32) + bc_ref[...]
    te_ref[...] = t.reshape(te_ref.shape)


def _final_body(ne_ref, p0_ref, p1_ref, w_ref, b_ref, z_ref):
    h = ne_ref[...] + p0_ref[...] + p1_ref[...]
    z = jnp.sum(h * w_ref[...], axis=1, keepdims=True) + b_ref[...]
    z_ref[...] = jax.nn.sigmoid(z)


_NC = 2
_NS = 16
_NW = _NC * _NS
_CB = 128
_GRP = 8


def _make_scatter_kernel(n_pad, total_chunks):
    base_c = total_chunks // _NW
    extra = total_chunks - base_c * _NW
    max_c = base_c + (1 if extra else 0)
    groups = base_c // _GRP
    tail = base_c - groups * _GRP
    rows_per_s = n_pad // _NS
    mesh = plsc.VectorSubcoreMesh(core_axis_name="c", subcore_axis_name="s",
                                  num_cores=_NC, num_subcores=_NS)

    @functools.partial(
        pl.kernel,
        out_type=jax.ShapeDtypeStruct((_NC, n_pad, 32), jnp.float32),
        mesh=mesh,
        scratch_types=[
            pltpu.VMEM((max_c, _CB), jnp.int32),
            pltpu.VMEM((max_c, _CB), jnp.int32),
            pltpu.VMEM((_GRP, _CB * 32), jnp.float32),
            pltpu.VMEM((_GRP, _CB, 32), jnp.float32),
            pltpu.VMEM_SHARED((n_pad, 32), jnp.float32),
            pltpu.SemaphoreType.DMA,
            pltpu.SemaphoreType.DMA,
        ],
        compiler_params=pltpu.CompilerParams(use_tc_tiling_on_sc=False),
    )
    def scatter_k(src_hbm, dst_hbm, te_hbm, tn_hbm, zeros_hbm, out_hbm,
                  sidx, didx, tev, rowsv, acc, gsem, ssem):
        c = lax.axis_index("c")
        s = lax.axis_index("s")
        wid = s * _NC + c
        start = wid * base_c + jnp.minimum(wid, extra)
        pltpu.sync_copy(zeros_hbm.at[pl.ds(s * rows_per_s, rows_per_s)],
                        acc.at[pl.ds(s * rows_per_s, rows_per_s)])

        @pl.when(wid < extra)
        def _stage_big():
            pltpu.sync_copy(src_hbm.at[pl.ds(start, max_c)], sidx)
            pltpu.sync_copy(dst_hbm.at[pl.ds(start, max_c)], didx)

        @pl.when(wid >= extra)
        def _stage_small():
            pltpu.sync_copy(src_hbm.at[pl.ds(start, base_c)],
                            sidx.at[pl.ds(0, base_c)])
            pltpu.sync_copy(dst_hbm.at[pl.ds(start, base_c)],
                            didx.at[pl.ds(0, base_c)])

        plsc.subcore_barrier()

        def do_chunk_loads(j, r):
            g = pltpu.async_copy(tn_hbm.at[sidx.at[j]], rowsv.at[r], gsem)
            t = pltpu.async_copy(
                te_hbm.at[pl.ds((start + j) * (_CB * 32), _CB * 32)],
                tev.at[r], gsem)
            return g, t

        def add_te(r):
            def add_body(i, carry):
                plsc.addupdate(rowsv.at[r, i, pl.ds(0, 16)],
                               tev[r, pl.ds(i * 32, 16)])
                plsc.addupdate(rowsv.at[r, i, pl.ds(16, 16)],
                               tev[r, pl.ds(i * 32 + 16, 16)])
                return carry
            lax.fori_loop(0, _CB, add_body, 0)

        def body(g, carry):
            j0 = g * _GRP
            loads = []
            for r in range(_GRP):
                loads.append(do_chunk_loads(j0 + r, r))
            stores = []
            for r in range(_GRP):
                for d in loads[r]:
                    d.wait()
                add_te(r)
                stores.append(pltpu.async_copy(
                    rowsv.at[r], acc.at[didx.at[j0 + r]], ssem, add=True))
            for d in stores:
                d.wait()
            return carry

        lax.fori_loop(0, groups, body, 0)

        if tail:
            j0 = groups * _GRP
            loads = [do_chunk_loads(j0 + r, r) for r in range(tail)]
            stores = []
            for r in range(tail):
                for d in loads[r]:
                    d.wait()
                add_te(r)
                stores.append(pltpu.async_copy(
                    rowsv.at[r], acc.at[didx.at[j0 + r]], ssem, add=True))
            for d in stores:
                d.wait()

        if extra:
            @pl.when(wid < extra)
            def _extra_chunk():
                g, t = do_chunk_loads(base_c, 0)
                g.wait()
                t.wait()
                add_te(0)
                pltpu.async_copy(rowsv.at[0], acc.at[didx.at[base_c]],
                                 ssem, add=True).wait()

        plsc.subcore_barrier()
        pltpu.sync_copy(acc.at[pl.ds(s * rows_per_s, rows_per_s)],
                        out_hbm.at[c, pl.ds(s * rows_per_s, rows_per_s)])

    return scatter_k


def _make_gather_kernel(n_nodes, p_pad):
    per_w = p_pad // _NW
    groups = per_w // 16
    mesh = plsc.VectorSubcoreMesh(core_axis_name="c", subcore_axis_name="s",
                                  num_cores=_NC, num_subcores=_NS)

    @functools.partial(
        pl.kernel,
        out_type=jax.ShapeDtypeStruct((p_pad,), jnp.float32),
        mesh=mesh,
        scratch_types=[
            pltpu.VMEM((n_nodes,), jnp.float32),
            pltpu.VMEM((per_w,), jnp.int32),
            pltpu.VMEM((per_w,), jnp.float32),
        ],
        compiler_params=pltpu.CompilerParams(needs_layout_passes=False),
    )
    def gather_k(z_hbm, pm_hbm, out_hbm, zv, idxv, outv):
        c = lax.axis_index("c")
        s = lax.axis_index("s")
        wid = s * _NC + c
        pltpu.sync_copy(z_hbm, zv)
        pltpu.sync_copy(pm_hbm.at[pl.ds(wid * per_w, per_w)], idxv)
        for g in range(groups):
            idx = idxv[pl.ds(g * 16, 16)]
            outv[pl.ds(g * 16, 16)] = plsc.load_gather(zv, [idx])
        pltpu.sync_copy(outv, out_hbm.at[pl.ds(wid * per_w, per_w)])

    return gather_k


def kernel(node_features, edge_index, edge_attr, post_mask,
           W_node, b_node, W_edge, b_edge, W_conv, b_conv, W_out, b_out):
    n, d_node = node_features.shape
    e = edge_attr.shape[0]
    d_edge = edge_attr.shape[1]
    h = W_node.shape[0]
    p = post_mask.shape[0]

    total_chunks = e // _CB
    n_pad = -(-(n + 1) // (8 * _NS)) * (8 * _NS)
    p_pad = -(-p // (16 * _NW)) * (16 * _NW)

    wnt = W_node.T
    wet = W_edge.T
    a_mat = W_conv[:, :h].T
    b_mat = W_conv[:, h:].T
    bn2 = b_node.reshape(1, h)
    be2 = b_edge.reshape(1, h)
    bc2 = b_conv.reshape(1, h)
    w2 = W_out.reshape(1, h)
    bo2 = b_out.reshape(1, 1)

    src = edge_index[0]
    dst = edge_index[1]
    ep8 = edge_attr.reshape(e // 8, 8 * d_edge)
    bd1 = jnp.kron(jnp.eye(8, dtype=jnp.float32), wet)
    bd2 = jnp.kron(jnp.eye(8, dtype=jnp.float32), b_mat)
    be8 = jnp.tile(b_edge, 8).reshape(1, 8 * h)
    bc8 = jnp.tile(b_conv, 8).reshape(1, 8 * h)
    pm = jnp.pad(post_mask, (0, p_pad - p))
    zeros_acc = jnp.zeros((n_pad, 32), jnp.float32)

    nb = 512
    ng = -(-n // nb)
    node_emb, tn = pl.pallas_call(
        _node_body,
        grid=(ng,),
        in_specs=[
            pl.BlockSpec((nb, d_node), lambda i: (i, 0)),
            pl.BlockSpec((d_node, h), lambda i: (0, 0)),
            pl.BlockSpec((1, h), lambda i: (0, 0)),
            pl.BlockSpec((h, h), lambda i: (0, 0)),
        ],
        out_specs=[pl.BlockSpec((nb, h), lambda i: (i, 0)),
                   pl.BlockSpec((nb, h), lambda i: (i, 0))],
        out_shape=[jax.ShapeDtypeStruct((n, h), jnp.float32),
                   jax.ShapeDtypeStruct((n, h), jnp.float32)],
    )(node_features, wnt, bn2, a_mat)

    eb = 800
    eg = (e // 8) // eb
    te = pl.pallas_call(
        _edge_body,
        grid=(eg,),
        in_specs=[
            pl.BlockSpec((eb, 8 * d_edge), lambda i: (i, 0)),
            pl.BlockSpec((8 * d_edge, 8 * h), lambda i: (0, 0)),
            pl.BlockSpec((1, 8 * h), lambda i: (0, 0)),
            pl.BlockSpec((8 * h, 8 * h), lambda i: (0, 0)),
            pl.BlockSpec((1, 8 * h), lambda i: (0, 0)),
        ],
        out_specs=pl.BlockSpec((2 * eb, 4 * h), lambda i: (i, 0)),
        out_shape=jax.ShapeDtypeStruct((e * h // (4 * h), 4 * h), jnp.float32),
    )(ep8, bd1, be8, bd2, bc8)

    partials = _make_scatter_kernel(n_pad, total_chunks)(
        src.reshape(total_chunks, _CB), dst.reshape(total_chunks, _CB),
        te.reshape(-1), tn, zeros_acc)

    fb = 1024
    fg = -(-n_pad // fb)
    z = pl.pallas_call(
        _final_body,
        grid=(fg,),
        in_specs=[
            pl.BlockSpec((fb, h), lambda i: (i, 0)),
            pl.BlockSpec((fb, h), lambda i: (i, 0)),
            pl.BlockSpec((fb, h), lambda i: (i, 0)),
            pl.BlockSpec((1, h), lambda i: (0, 0)),
            pl.BlockSpec((1, 1), lambda i: (0, 0)),
        ],
        out_specs=pl.BlockSpec((fb, 1), lambda i: (i, 0)),
        out_shape=jax.ShapeDtypeStruct((n, 1), jnp.float32),
    )(node_emb, partials[0], partials[1], w2, bo2)

    out = _make_gather_kernel(n, p_pad)(z.reshape(n), pm)
    return out[:p]
